# baseline (speedup 1.0000x reference)
import functools
import jax
import jax.numpy as jnp
from jax import lax
from jax.experimental import pallas as pl
from jax.experimental.pallas import tpu as pltpu
from jax.experimental.pallas import tpu_sc as plsc

N = 4096
MOL = 16
MH = 64
UH = 64
HEADS = 2
HU = HEADS * UH
CUTOFF = 0.25
CUT2 = CUTOFF * CUTOFF
SELF_DIM = 2 + MOL + 1 + 1
OUT_DIM = 2 + 1 + MOL + 1
K = 256
PAY = 32

NC = 2
NS = 16
NW = NC * NS
RW = N // NW
LANES = 16
GR = 32
NG = RW // GR


def _nbr_body(xs_hbm, ys_hbm, ang_hbm, molt_hbm, gen_hbm, deg_hbm,
              sang_hbm, cang_hbm,
              nbr_hbm, dx_hbm, dy_hbm, da_hbm, pay_hbm,
              xs_v, ys_v, ang_v, nbr_g, dx_g, dy_g, da_g,
              idx_v, tbl_v, out_v):
    cid = lax.axis_index("c")
    sid = lax.axis_index("s")
    wid = sid * NC + cid
    base = wid * RW
    pltpu.sync_copy(xs_hbm, xs_v)
    pltpu.sync_copy(ys_hbm, ys_v)
    pltpu.sync_copy(ang_hbm, ang_v)

    zero16i = jnp.zeros((LANES,), jnp.int32)
    zero16f = jnp.zeros((LANES,), jnp.float32)
    lane_iota = lax.iota(jnp.int32, LANES)

    def group_body(g, carry0):
        def zb(t, carry):
            sl = pl.ds(t * LANES, LANES)
            nbr_g[sl] = zero16i
            dx_g[sl] = zero16f
            dy_g[sl] = zero16f
            da_g[sl] = zero16f
            return carry

        lax.fori_loop(0, GR * K // LANES, zb, 0)

        def row_body(rr, carry):
            r = g * GR + rr
            i = base + r
            iv = jnp.full((LANES,), i, jnp.int32)
            xi = plsc.load_gather(xs_v, [iv])
            yi = plsc.load_gather(ys_v, [iv])
            ai = plsc.load_gather(ang_v, [iv])
            rowbase = rr * K

            def cb(cc, cnt):
                off = cc * LANES
                jv = lane_iota + off
                xj = xs_v[pl.ds(off, LANES)]
                yj = ys_v[pl.ds(off, LANES)]
                dxv = xj - xi
                dyv = yj - yi
                d2 = dxv * dxv + dyv * dyv
                m = jnp.logical_and(d2 <= CUT2, jv != i)
                mi = m.astype(jnp.int32)
                pos = jnp.minimum(cnt + plsc.cumsum(mi) - 1, K - 1) + rowbase
                aj = ang_v[pl.ds(off, LANES)]
                plsc.store_scatter(nbr_g, [pos], jv, mask=m)
                plsc.store_scatter(dx_g, [pos], dxv, mask=m)
                plsc.store_scatter(dy_g, [pos], dyv, mask=m)
                plsc.store_scatter(da_g, [pos], aj - ai, mask=m)
                return cnt + jnp.sum(mi)

            lax.fori_loop(0, N // LANES, cb, jnp.int32(0), unroll=4)
            return carry

        lax.fori_loop(0, GR, row_body, 0)
        gbase = (base + g * GR) * K
        sl = pl.ds(gbase, GR * K)
        pltpu.sync_copy(nbr_g, nbr_hbm.at[sl])
        pltpu.sync_copy(dx_g, dx_hbm.at[sl])
        pltpu.sync_copy(dy_g, dy_hbm.at[sl])
        pltpu.sync_copy(da_g, da_hbm.at[sl])
        return carry0

    lax.fori_loop(0, NG, group_body, 0)

    ebase = base * K
    pltpu.sync_copy(nbr_hbm.at[pl.ds(ebase, RW * K)], idx_v)

    col_srcs = [molt_hbm.at[pl.ds(c * N, N)] for c in range(MOL)]
    col_srcs += [gen_hbm, deg_hbm, sang_hbm, cang_hbm]

    for col, src in enumerate(col_srcs):
        pltpu.sync_copy(src, tbl_v)

        def gcb(t, carry):
            sl = pl.ds(t * LANES, LANES)
            out_v[sl] = plsc.load_gather(tbl_v, [idx_v[sl]])
            return carry

        lax.fori_loop(0, RW * K // LANES, gcb, 0)
        pltpu.sync_copy(out_v, pay_hbm.at[col, pl.ds(ebase, RW * K)])

    zf = jnp.zeros((LANES,), jnp.float32)

    def zpb(t, carry):
        out_v[pl.ds(t * LANES, LANES)] = zf
        return carry

    lax.fori_loop(0, RW * K // LANES, zpb, 0)
    for col in range(NFC, PAYR):
        pltpu.sync_copy(out_v, pay_hbm.at[col, pl.ds(ebase, RW * K)])


def _nbr_call(xs, ys, ang, mol_t, gen, deg, sang, cang):
    f = pl.kernel(
        _nbr_body,
        out_type=(
            jax.ShapeDtypeStruct((N * K,), jnp.int32),
            jax.ShapeDtypeStruct((N * K,), jnp.float32),
            jax.ShapeDtypeStruct((N * K,), jnp.float32),
            jax.ShapeDtypeStruct((N * K,), jnp.float32),
            jax.ShapeDtypeStruct((PAYR, N * K), jnp.float32),
        ),
        mesh=plsc.VectorSubcoreMesh(core_axis_name="c", subcore_axis_name="s",
                                    num_cores=NC, num_subcores=NS),
        scratch_types=[
            pltpu.VMEM((N,), jnp.float32),
            pltpu.VMEM((N,), jnp.float32),
            pltpu.VMEM((N,), jnp.float32),
            pltpu.VMEM((GR * K,), jnp.int32),
            pltpu.VMEM((GR * K,), jnp.float32),
            pltpu.VMEM((GR * K,), jnp.float32),
            pltpu.VMEM((GR * K,), jnp.float32),
            pltpu.VMEM((RW * K,), jnp.int32),
            pltpu.VMEM((N,), jnp.float32),
            pltpu.VMEM((RW * K,), jnp.float32),
        ],
        compiler_params=pltpu.CompilerParams(use_tc_tiling_on_sc=False,
                                             needs_layout_passes=False),
    )
    return f(xs, ys, ang, mol_t, gen, deg, sang, cang)


NFC = 20
PAYR = 24


DT = 512


def _deg_body(xt_ref, yt_ref, xa_ref, ya_ref, deg_ref):
    xt = xt_ref[...]
    yt = yt_ref[...]
    xa = xa_ref[...]
    ya = ya_ref[...]
    dxm = xt - xa
    dym = yt - ya
    m = (dxm * dxm + dym * dym) <= CUT2
    deg_ref[...] = jnp.sum(m.astype(jnp.float32), axis=1, keepdims=True) - 1.0


def _deg_call(xs, ys):
    return pl.pallas_call(
        _deg_body,
        grid=(N // DT,),
        in_specs=[
            pl.BlockSpec((DT, 1), lambda i: (i, 0)),
            pl.BlockSpec((DT, 1), lambda i: (i, 0)),
            pl.BlockSpec((1, N), lambda i: (0, 0)),
            pl.BlockSpec((1, N), lambda i: (0, 0)),
        ],
        out_specs=pl.BlockSpec((DT, 1), lambda i: (i, 0)),
        out_shape=jax.ShapeDtypeStruct((N, 1), jnp.float32),
    )(xs[:, None], ys[:, None], xs[None, :], ys[None, :])


def _sincos_body(a_ref, s_ref, c_ref):
    a = a_ref[...]
    s_ref[...] = jnp.sin(a)
    c_ref[...] = jnp.cos(a)


def _sincos_call(ang):
    s, c = pl.pallas_call(
        _sincos_body,
        out_shape=(jax.ShapeDtypeStruct((32, 128), jnp.float32),
                   jax.ShapeDtypeStruct((32, 128), jnp.float32)),
    )(ang.reshape(32, 128))
    return s.reshape(N), c.reshape(N)


TN = 32
E = TN * K


def _bc(s, w):
    return lax.broadcast_in_dim(s, (TN, K, w.shape[-1]), (0, 1)) * w[None]


def _dott(a_t, b):
    return lax.dot_general(a_t, b, (((0,), (0,)), ((), ())),
                           preferred_element_type=jnp.float32)


def _tc_body(pay_ref, nf_ref, deg_ref, sang_ref, cang_ref,
             dx_ref, dy_ref, da_ref,
             w1at_ref, w1a_ref, w1s_ref, b1_ref, w2_ref, b2_ref, w3_ref,
             b3_ref, wkvpt_ref, wem2_ref, bkv_ref, wes_ref, wq_ref, bq_ref,
             wskip_ref, bskip_ref, hw1_ref, hb1_ref, hw2_ref, hb2_ref,
             hw3_ref, hb3_ref, out_ref):
    PT = pay_ref[...]
    ni = nf_ref[...]
    dx = dx_ref[...]
    dy = dy_ref[...]
    da = da_ref[...]

    r = jnp.sqrt(jnp.maximum(dx * dx + dy * dy, 1e-12))
    sda = jnp.sin(da)
    cda = jnp.cos(da)

    pre = (_dott(PT, w1at_ref[...]).reshape(TN, K, MH)
           + (b1_ref[...] - ni @ w1a_ref[...])[:, None, :]
           + _bc(dx, w1s_ref[0:1]) + _bc(dy, w1s_ref[1:2])
           + _bc(r, w1s_ref[2:3]) + _bc(sda, w1s_ref[3:4])
           + _bc(cda, w1s_ref[4:5]))
    h = jnp.maximum(pre, 0.0).reshape(E, MH)
    h = jnp.maximum(h @ w2_ref[...] + b2_ref[...], 0.0)
    msg = jnp.maximum(h @ w3_ref[...] + b3_ref[...], 0.0)

    sfi = jnp.concatenate(
        [sang_ref[...], cang_ref[...], ni[:, 0:16], ni[:, 16:17],
         deg_ref[...]], axis=-1)
    qi = sfi @ wq_ref[...] + bq_ref[...]
    eself = sfi @ wes_ref[...]

    ib = bkv_ref[...] + jnp.concatenate([eself, eself], axis=-1)
    KV2 = ((_dott(PT, wkvpt_ref[...]) + msg @ wem2_ref[...])
           .reshape(TN, K, 4 * UH) + ib[:, None, :])
    ve = KV2[..., 2 * UH:]

    kje = KV2[..., :2 * UH] * qi[:, None, :]
    scale = 1.0 / (float(UH) ** 0.5)
    a_h = [jnp.sum(kje[..., h_ * UH:(h_ + 1) * UH], axis=-1) * scale
           for h_ in range(HEADS)]

    degi = deg_ref[...]
    slot = lax.broadcasted_iota(jnp.int32, (TN, K), 1).astype(jnp.float32)
    valid = slot < jnp.minimum(degi, float(K))

    outs = []
    for h_ in range(HEADS):
        ah = jnp.where(valid, a_h[h_], -1e30)
        am = jnp.max(ah, axis=1, keepdims=True)
        am = jnp.where(degi > 0.0, am, 0.0)
        ea = jnp.where(valid, jnp.exp(ah - am), 0.0)
        den = jnp.sum(ea, axis=1, keepdims=True)
        inv = 1.0 / (den + 1e-16)
        eab = lax.broadcast_in_dim(ea, (TN, K, UH), (0, 1))
        oh = jnp.sum(ve[..., h_ * UH:(h_ + 1) * UH] * eab, axis=1)
        outs.append(oh * inv)

    out = (outs[0] + outs[1]) * 0.5 + sfi @ wskip_ref[...] + bskip_ref[...]
    hh = jnp.maximum(out @ hw1_ref[...] + hb1_ref[...], 0.0)
    hh = jnp.maximum(hh @ hw2_ref[...] + hb2_ref[...], 0.0)
    out_ref[...] = hh @ hw3_ref[...] + hb3_ref[...]


def _tc_call(payload_t, nodefeat0, deg, sang, cang, dxp, dyp, dap, *weights):
    wspecs = [pl.BlockSpec(w.shape, lambda i, nd=w.ndim: (0,) * nd)
              for w in weights]
    return pl.pallas_call(
        _tc_body,
        grid=(N // TN,),
        in_specs=[
            pl.BlockSpec((PAYR, E), lambda i: (0, i)),
            pl.BlockSpec((TN, 24), lambda i: (i, 0)),
            pl.BlockSpec((TN, 1), lambda i: (i, 0)),
            pl.BlockSpec((TN, 1), lambda i: (i, 0)),
            pl.BlockSpec((TN, 1), lambda i: (i, 0)),
            pl.BlockSpec((TN, K), lambda i: (i, 0)),
            pl.BlockSpec((TN, K), lambda i: (i, 0)),
            pl.BlockSpec((TN, K), lambda i: (i, 0)),
            *wspecs,
        ],
        out_specs=pl.BlockSpec((TN, PAY), lambda i: (i, 0)),
        out_shape=jax.ShapeDtypeStruct((N, PAY), jnp.float32),
    )(payload_t, nodefeat0, deg, sang, cang, dxp, dyp, dap, *weights)


def kernel(x, angle, molecules, generation, msg_W1, msg_b1, msg_W2, msg_b2,
           msg_W3, msg_b3, Wq, bq, Wk, bk, Wv, bv, We, Wskip, bskip,
           head_W1, head_b1, head_W2, head_b2, head_W3, head_b3):
    f32 = jnp.float32
    xs = jnp.asarray(x[:, 0], f32)
    ys = jnp.asarray(x[:, 1], f32)
    ang = jnp.asarray(angle[:, 0], f32)

    deg2 = _deg_call(xs, ys)
    deg = deg2[:, 0]
    sang, cang = _sincos_call(ang)
    mol_t = molecules.T.reshape(-1)
    gen = generation[:, 0]

    nbr_flat, dxf, dyf, daf, payload_t = _nbr_call(
        xs, ys, ang, mol_t, gen, deg, sang, cang)

    nodefeat0 = jnp.concatenate(
        [molecules, generation, jnp.zeros((N, 7), f32)], axis=1)

    w1at = jnp.zeros((PAYR, MH), f32).at[0:16].set(msg_W1[5:21])
    w1a = jnp.zeros((24, MH), f32).at[0:16].set(msg_W1[5:21])
    w1s = jnp.zeros((8, MH), f32).at[0:5].set(msg_W1[0:5])
    wkv = jnp.concatenate([Wk, Wv], axis=1)
    wkvpt = jnp.zeros((PAYR, 2 * HU), f32).at[0:16].set(wkv[2:18])
    wkvpt = (wkvpt.at[16].set(wkv[18]).at[17].set(wkv[19])
             .at[18].set(wkv[0]).at[19].set(wkv[1]))
    bkv = jnp.concatenate([bk, bv])[None, :]
    wem = We[:MH]
    wem2 = jnp.concatenate([wem, wem], axis=1)
    wes = We[MH:]
    hw3p = jnp.concatenate([head_W3, jnp.zeros((UH, PAY - OUT_DIM), f32)],
                           axis=1)
    hb3p = jnp.concatenate([head_b3, jnp.zeros((PAY - OUT_DIM,), f32)])[None, :]

    upd = _tc_call(payload_t, nodefeat0, deg2, sang[:, None],
                   cang[:, None],
                   dxf.reshape(N, K), dyf.reshape(N, K), daf.reshape(N, K),
                   w1at, w1a, w1s, msg_b1[None, :], msg_W2, msg_b2[None, :],
                   msg_W3, msg_b3[None, :], wkvpt, wem2, bkv, wes,
                   Wq, bq[None, :], Wskip, bskip[None, :],
                   head_W1, head_b1[None, :], head_W2, head_b2[None, :],
                   hw3p, hb3p)

    return (upd[:, 0:2], upd[:, 2:3], upd[:, 3:3 + MOL],
            upd[:, 3 + MOL:4 + MOL])


# speedup vs baseline: 1.0245x; 1.0245x over previous
import functools
import jax
import jax.numpy as jnp
from jax import lax
from jax.experimental import pallas as pl
from jax.experimental.pallas import tpu as pltpu
from jax.experimental.pallas import tpu_sc as plsc

N = 4096
MOL = 16
MH = 64
UH = 64
HEADS = 2
HU = HEADS * UH
CUTOFF = 0.25
CUT2 = CUTOFF * CUTOFF
SELF_DIM = 2 + MOL + 1 + 1
OUT_DIM = 2 + 1 + MOL + 1
K = 256
PAY = 32

NC = 2
NS = 16
NW = NC * NS
RW = N // NW
LANES = 16
GR = 32
NG = RW // GR


def _nbr_body(xs_hbm, ys_hbm, ang_hbm, molt_hbm, gen_hbm, deg_hbm,
              sang_hbm, cang_hbm,
              nbr_hbm, dx_hbm, dy_hbm, da_hbm, pay_hbm,
              xs_v, ys_v, ang_v, nbr_g, dx_g, dy_g, da_g,
              idx_v, tbl_v, out_v):
    cid = lax.axis_index("c")
    sid = lax.axis_index("s")
    wid = sid * NC + cid
    base = wid * RW
    pltpu.sync_copy(xs_hbm, xs_v)
    pltpu.sync_copy(ys_hbm, ys_v)
    pltpu.sync_copy(ang_hbm, ang_v)

    zero16i = jnp.zeros((LANES,), jnp.int32)
    zero16f = jnp.zeros((LANES,), jnp.float32)
    lane_iota = lax.iota(jnp.int32, LANES)

    def group_body(g, carry0):
        def zb(t, carry):
            sl = pl.ds(t * LANES, LANES)
            nbr_g[sl] = zero16i
            dx_g[sl] = zero16f
            dy_g[sl] = zero16f
            da_g[sl] = zero16f
            return carry

        lax.fori_loop(0, GR * K // LANES, zb, 0)

        def row_body(rr, carry):
            r = g * GR + rr
            i = base + r
            iv = jnp.full((LANES,), i, jnp.int32)
            xi = plsc.load_gather(xs_v, [iv])
            yi = plsc.load_gather(ys_v, [iv])
            ai = plsc.load_gather(ang_v, [iv])
            rowbase = rr * K

            def cb(cc, cnt):
                off = cc * LANES
                jv = lane_iota + off
                xj = xs_v[pl.ds(off, LANES)]
                yj = ys_v[pl.ds(off, LANES)]
                dxv = xj - xi
                dyv = yj - yi
                d2 = dxv * dxv + dyv * dyv
                m = jnp.logical_and(d2 <= CUT2, jv != i)
                mi = m.astype(jnp.int32)
                pos = jnp.minimum(cnt + plsc.cumsum(mi) - 1, K - 1) + rowbase
                aj = ang_v[pl.ds(off, LANES)]
                plsc.store_scatter(nbr_g, [pos], jv, mask=m)
                plsc.store_scatter(dx_g, [pos], dxv, mask=m)
                plsc.store_scatter(dy_g, [pos], dyv, mask=m)
                plsc.store_scatter(da_g, [pos], aj - ai, mask=m)
                return cnt + jnp.sum(mi)

            lax.fori_loop(0, N // LANES, cb, jnp.int32(0), unroll=4)
            return carry

        lax.fori_loop(0, GR, row_body, 0)
        gbase = (base + g * GR) * K
        sl = pl.ds(gbase, GR * K)
        pltpu.sync_copy(nbr_g, nbr_hbm.at[sl])
        pltpu.sync_copy(dx_g, dx_hbm.at[sl])
        pltpu.sync_copy(dy_g, dy_hbm.at[sl])
        pltpu.sync_copy(da_g, da_hbm.at[sl])
        return carry0

    lax.fori_loop(0, NG, group_body, 0)

    ebase = base * K
    pltpu.sync_copy(nbr_hbm.at[pl.ds(ebase, RW * K)], idx_v)

    col_srcs = [molt_hbm.at[pl.ds(c * N, N)] for c in range(MOL)]
    col_srcs += [gen_hbm, deg_hbm, sang_hbm, cang_hbm]

    for col, src in enumerate(col_srcs):
        pltpu.sync_copy(src, tbl_v)

        def gcb(t, carry):
            sl = pl.ds(t * LANES, LANES)
            out_v[sl] = plsc.load_gather(tbl_v, [idx_v[sl]])
            return carry

        lax.fori_loop(0, RW * K // LANES, gcb, 0)
        pltpu.sync_copy(out_v, pay_hbm.at[col, pl.ds(ebase, RW * K)])

    zf = jnp.zeros((LANES,), jnp.float32)

    def zpb(t, carry):
        out_v[pl.ds(t * LANES, LANES)] = zf
        return carry

    lax.fori_loop(0, RW * K // LANES, zpb, 0)
    for col in range(NFC, PAYR):
        pltpu.sync_copy(out_v, pay_hbm.at[col, pl.ds(ebase, RW * K)])


def _nbr_call(xs, ys, ang, mol_t, gen, deg, sang, cang):
    f = pl.kernel(
        _nbr_body,
        out_type=(
            jax.ShapeDtypeStruct((N * K,), jnp.int32),
            jax.ShapeDtypeStruct((N * K,), jnp.float32),
            jax.ShapeDtypeStruct((N * K,), jnp.float32),
            jax.ShapeDtypeStruct((N * K,), jnp.float32),
            jax.ShapeDtypeStruct((PAYR, N * K), jnp.float32),
        ),
        mesh=plsc.VectorSubcoreMesh(core_axis_name="c", subcore_axis_name="s",
                                    num_cores=NC, num_subcores=NS),
        scratch_types=[
            pltpu.VMEM((N,), jnp.float32),
            pltpu.VMEM((N,), jnp.float32),
            pltpu.VMEM((N,), jnp.float32),
            pltpu.VMEM((GR * K,), jnp.int32),
            pltpu.VMEM((GR * K,), jnp.float32),
            pltpu.VMEM((GR * K,), jnp.float32),
            pltpu.VMEM((GR * K,), jnp.float32),
            pltpu.VMEM((RW * K,), jnp.int32),
            pltpu.VMEM((N,), jnp.float32),
            pltpu.VMEM((RW * K,), jnp.float32),
        ],
        compiler_params=pltpu.CompilerParams(use_tc_tiling_on_sc=False,
                                             needs_layout_passes=False),
    )
    return f(xs, ys, ang, mol_t, gen, deg, sang, cang)


NFC = 20
PAYR = 24


DT = 512


def _deg_body(xt_ref, yt_ref, xa_ref, ya_ref, deg_ref):
    xt = xt_ref[...]
    yt = yt_ref[...]
    xa = xa_ref[...]
    ya = ya_ref[...]
    dxm = xt - xa
    dym = yt - ya
    m = (dxm * dxm + dym * dym) <= CUT2
    deg_ref[...] = jnp.sum(m.astype(jnp.float32), axis=1, keepdims=True) - 1.0


def _deg_call(xs, ys):
    return pl.pallas_call(
        _deg_body,
        grid=(N // DT,),
        in_specs=[
            pl.BlockSpec((DT, 1), lambda i: (i, 0)),
            pl.BlockSpec((DT, 1), lambda i: (i, 0)),
            pl.BlockSpec((1, N), lambda i: (0, 0)),
            pl.BlockSpec((1, N), lambda i: (0, 0)),
        ],
        out_specs=pl.BlockSpec((DT, 1), lambda i: (i, 0)),
        out_shape=jax.ShapeDtypeStruct((N, 1), jnp.float32),
    )(xs[:, None], ys[:, None], xs[None, :], ys[None, :])


def _sincos_body(a_ref, s_ref, c_ref):
    a = a_ref[...]
    s_ref[...] = jnp.sin(a)
    c_ref[...] = jnp.cos(a)


def _sincos_call(ang):
    s, c = pl.pallas_call(
        _sincos_body,
        out_shape=(jax.ShapeDtypeStruct((32, 128), jnp.float32),
                   jax.ShapeDtypeStruct((32, 128), jnp.float32)),
    )(ang.reshape(32, 128))
    return s.reshape(N), c.reshape(N)


TN = 32
E = TN * K


def _bc(s, w):
    return lax.broadcast_in_dim(s, (TN, K, w.shape[-1]), (0, 1)) * w[None]


def _dott(a_t, b):
    return lax.dot_general(a_t, b, (((0,), (0,)), ((), ())),
                           preferred_element_type=jnp.float32)


def _tc_body(pay_ref, nf_ref, deg_ref, sang_ref, cang_ref,
             dx_ref, dy_ref, da_ref,
             w1at_ref, w1a_ref, w1s_ref, b1_ref, w2_ref, b2_ref, w3_ref,
             b3_ref, wkvpt_ref, wem2_ref, bkv_ref, wes_ref, wq_ref, bq_ref,
             wskip_ref, bskip_ref, hw1_ref, hb1_ref, hw2_ref, hb2_ref,
             hw3_ref, hb3_ref, out_ref):
    PT = pay_ref[...]
    ni = nf_ref[...]
    dx = dx_ref[...]
    dy = dy_ref[...]
    da = da_ref[...]

    r = jnp.sqrt(jnp.maximum(dx * dx + dy * dy, 1e-12))
    sda = jnp.sin(da)
    cda = jnp.cos(da)

    bf16 = jnp.bfloat16
    PTb = PT.astype(bf16)
    pre = (_dott(PTb, w1at_ref[...]).reshape(TN, K, MH)
           + (b1_ref[...] - ni @ w1a_ref[...])[:, None, :]
           + _bc(dx, w1s_ref[0:1]) + _bc(dy, w1s_ref[1:2])
           + _bc(r, w1s_ref[2:3]) + _bc(sda, w1s_ref[3:4])
           + _bc(cda, w1s_ref[4:5]))
    h = jnp.maximum(pre, 0.0).reshape(E, MH).astype(bf16)
    h = jnp.maximum(lax.dot_general(h, w2_ref[...], (((1,), (0,)), ((), ())),
                                    preferred_element_type=jnp.float32)
                    + b2_ref[...], 0.0).astype(bf16)
    msg = jnp.maximum(lax.dot_general(h, w3_ref[...], (((1,), (0,)), ((), ())),
                                      preferred_element_type=jnp.float32)
                      + b3_ref[...], 0.0)

    sfi = jnp.concatenate(
        [sang_ref[...], cang_ref[...], ni[:, 0:16], ni[:, 16:17],
         deg_ref[...]], axis=-1)
    qi = sfi @ wq_ref[...] + bq_ref[...]
    eself = sfi @ wes_ref[...]

    ib = bkv_ref[...] + jnp.concatenate([eself, eself], axis=-1)
    KV2 = ((_dott(PTb, wkvpt_ref[...])
            + lax.dot_general(msg.astype(bf16), wem2_ref[...],
                              (((1,), (0,)), ((), ())),
                              preferred_element_type=jnp.float32))
           .reshape(TN, K, 4 * UH) + ib[:, None, :])
    ve = KV2[..., 2 * UH:]

    kje = KV2[..., :2 * UH] * qi[:, None, :]
    scale = 1.0 / (float(UH) ** 0.5)
    a_h = [jnp.sum(kje[..., h_ * UH:(h_ + 1) * UH], axis=-1) * scale
           for h_ in range(HEADS)]

    degi = deg_ref[...]
    slot = lax.broadcasted_iota(jnp.int32, (TN, K), 1).astype(jnp.float32)
    valid = slot < jnp.minimum(degi, float(K))

    outs = []
    for h_ in range(HEADS):
        ah = jnp.where(valid, a_h[h_], -1e30)
        am = jnp.max(ah, axis=1, keepdims=True)
        am = jnp.where(degi > 0.0, am, 0.0)
        ea = jnp.where(valid, jnp.exp(ah - am), 0.0)
        den = jnp.sum(ea, axis=1, keepdims=True)
        inv = 1.0 / (den + 1e-16)
        eab = lax.broadcast_in_dim(ea, (TN, K, UH), (0, 1))
        oh = jnp.sum(ve[..., h_ * UH:(h_ + 1) * UH] * eab, axis=1)
        outs.append(oh * inv)

    out = (outs[0] + outs[1]) * 0.5 + sfi @ wskip_ref[...] + bskip_ref[...]
    hh = jnp.maximum(out @ hw1_ref[...] + hb1_ref[...], 0.0)
    hh = jnp.maximum(hh @ hw2_ref[...] + hb2_ref[...], 0.0)
    out_ref[...] = hh @ hw3_ref[...] + hb3_ref[...]


def _tc_call(payload_t, nodefeat0, deg, sang, cang, dxp, dyp, dap, *weights):
    wspecs = [pl.BlockSpec(w.shape, lambda i, nd=w.ndim: (0,) * nd)
              for w in weights]
    return pl.pallas_call(
        _tc_body,
        grid=(N // TN,),
        in_specs=[
            pl.BlockSpec((PAYR, E), lambda i: (0, i)),
            pl.BlockSpec((TN, 24), lambda i: (i, 0)),
            pl.BlockSpec((TN, 1), lambda i: (i, 0)),
            pl.BlockSpec((TN, 1), lambda i: (i, 0)),
            pl.BlockSpec((TN, 1), lambda i: (i, 0)),
            pl.BlockSpec((TN, K), lambda i: (i, 0)),
            pl.BlockSpec((TN, K), lambda i: (i, 0)),
            pl.BlockSpec((TN, K), lambda i: (i, 0)),
            *wspecs,
        ],
        out_specs=pl.BlockSpec((TN, PAY), lambda i: (i, 0)),
        out_shape=jax.ShapeDtypeStruct((N, PAY), jnp.float32),
    )(payload_t, nodefeat0, deg, sang, cang, dxp, dyp, dap, *weights)


def kernel(x, angle, molecules, generation, msg_W1, msg_b1, msg_W2, msg_b2,
           msg_W3, msg_b3, Wq, bq, Wk, bk, Wv, bv, We, Wskip, bskip,
           head_W1, head_b1, head_W2, head_b2, head_W3, head_b3):
    f32 = jnp.float32
    xs = jnp.asarray(x[:, 0], f32)
    ys = jnp.asarray(x[:, 1], f32)
    ang = jnp.asarray(angle[:, 0], f32)

    deg2 = _deg_call(xs, ys)
    deg = deg2[:, 0]
    sang, cang = _sincos_call(ang)
    mol_t = molecules.T.reshape(-1)
    gen = generation[:, 0]

    nbr_flat, dxf, dyf, daf, payload_t = _nbr_call(
        xs, ys, ang, mol_t, gen, deg, sang, cang)

    nodefeat0 = jnp.concatenate(
        [molecules, generation, jnp.zeros((N, 7), f32)], axis=1)

    w1at = jnp.zeros((PAYR, MH), f32).at[0:16].set(msg_W1[5:21])
    w1a = jnp.zeros((24, MH), f32).at[0:16].set(msg_W1[5:21])
    w1s = jnp.zeros((8, MH), f32).at[0:5].set(msg_W1[0:5])
    wkv = jnp.concatenate([Wk, Wv], axis=1)
    wkvpt = jnp.zeros((PAYR, 2 * HU), f32).at[0:16].set(wkv[2:18])
    wkvpt = (wkvpt.at[16].set(wkv[18]).at[17].set(wkv[19])
             .at[18].set(wkv[0]).at[19].set(wkv[1]))
    bkv = jnp.concatenate([bk, bv])[None, :]
    wem = We[:MH]
    wem2 = jnp.concatenate([wem, wem], axis=1)
    wes = We[MH:]
    hw3p = jnp.concatenate([head_W3, jnp.zeros((UH, PAY - OUT_DIM), f32)],
                           axis=1)
    hb3p = jnp.concatenate([head_b3, jnp.zeros((PAY - OUT_DIM,), f32)])[None, :]

    bf16 = jnp.bfloat16
    upd = _tc_call(payload_t, nodefeat0, deg2, sang[:, None],
                   cang[:, None],
                   dxf.reshape(N, K), dyf.reshape(N, K), daf.reshape(N, K),
                   w1at.astype(bf16), w1a, w1s, msg_b1[None, :],
                   msg_W2.astype(bf16), msg_b2[None, :],
                   msg_W3.astype(bf16), msg_b3[None, :],
                   wkvpt.astype(bf16), wem2.astype(bf16), bkv, wes,
                   Wq, bq[None, :], Wskip, bskip[None, :],
                   head_W1, head_b1[None, :], head_W2, head_b2[None, :],
                   hw3p, hb3p)

    return (upd[:, 0:2], upd[:, 2:3], upd[:, 3:3 + MOL],
            upd[:, 3 + MOL:4 + MOL])


# speedup vs baseline: 1.0252x; 1.0007x over previous
import jax
import jax.numpy as jnp
from jax import lax
from jax.experimental import pallas as pl
from jax.experimental.pallas import tpu as pltpu
from jax.experimental.pallas import tpu_sc as plsc

N = 4096
MOL = 16
MH = 64
UH = 64
HEADS = 2
HU = HEADS * UH
CUTOFF = 0.25
CUT2 = CUTOFF * CUTOFF
SELF_DIM = 2 + MOL + 1 + 1
OUT_DIM = 2 + 1 + MOL + 1
K = 256
PAY = 32

NC = 2
NS = 16
NW = NC * NS
RW = N // NW
LANES = 16
GR = 32
NG = RW // GR
NFC = 20
PAYR = 24


def _nbr_body(xs_hbm, ys_hbm, ang_hbm, molt_hbm, gen_hbm, deg_hbm,
              sang_hbm, cang_hbm,
              nbr_hbm, dx_hbm, dy_hbm, da_hbm, pay_hbm,
              xs_v, ys_v, ang_v, nbr_g, dx_g, dy_g, da_g,
              idx_v, tbl_v, out_v):
    cid = lax.axis_index("c")
    sid = lax.axis_index("s")
    wid = sid * NC + cid
    base = wid * RW
    pltpu.sync_copy(xs_hbm, xs_v)
    pltpu.sync_copy(ys_hbm, ys_v)
    pltpu.sync_copy(ang_hbm, ang_v)

    zero16i = jnp.zeros((LANES,), jnp.int32)
    zero16f = jnp.zeros((LANES,), jnp.float32)
    lane_iota = lax.iota(jnp.int32, LANES)

    def group_body(g, carry0):
        def zb(t, carry):
            sl = pl.ds(t * LANES, LANES)
            nbr_g[sl] = zero16i
            dx_g[sl] = zero16f
            dy_g[sl] = zero16f
            da_g[sl] = zero16f
            return carry

        lax.fori_loop(0, GR * K // LANES, zb, 0)

        def row_body(rr, carry):
            r = g * GR + rr
            i = base + r
            iv = jnp.full((LANES,), i, jnp.int32)
            xi = plsc.load_gather(xs_v, [iv])
            yi = plsc.load_gather(ys_v, [iv])
            ai = plsc.load_gather(ang_v, [iv])
            rowbase = rr * K

            def cb(cc, cnt):
                off = cc * LANES
                jv = lane_iota + off
                xj = xs_v[pl.ds(off, LANES)]
                yj = ys_v[pl.ds(off, LANES)]
                dxv = xj - xi
                dyv = yj - yi
                d2 = dxv * dxv + dyv * dyv
                m = jnp.logical_and(d2 <= CUT2, jv != i)
                mi = m.astype(jnp.int32)
                pos = jnp.minimum(cnt + plsc.cumsum(mi) - 1, K - 1) + rowbase
                aj = ang_v[pl.ds(off, LANES)]
                plsc.store_scatter(nbr_g, [pos], jv, mask=m)
                plsc.store_scatter(dx_g, [pos], dxv, mask=m)
                plsc.store_scatter(dy_g, [pos], dyv, mask=m)
                plsc.store_scatter(da_g, [pos], aj - ai, mask=m)
                return cnt + jnp.sum(mi)

            lax.fori_loop(0, N // LANES, cb, jnp.int32(0), unroll=4)
            return carry

        lax.fori_loop(0, GR, row_body, 0)
        gbase = (base + g * GR) * K
        sl = pl.ds(gbase, GR * K)
        pltpu.sync_copy(nbr_g, nbr_hbm.at[sl])
        pltpu.sync_copy(dx_g, dx_hbm.at[sl])
        pltpu.sync_copy(dy_g, dy_hbm.at[sl])
        pltpu.sync_copy(da_g, da_hbm.at[sl])
        return carry0

    lax.fori_loop(0, NG, group_body, 0)

    ebase = base * K
    pltpu.sync_copy(nbr_hbm.at[pl.ds(ebase, RW * K)], idx_v)

    col_srcs = [molt_hbm.at[pl.ds(c * N, N)] for c in range(MOL)]
    col_srcs += [gen_hbm, deg_hbm, sang_hbm, cang_hbm]

    for col, src in enumerate(col_srcs):
        pltpu.sync_copy(src, tbl_v)

        def gcb(t, carry):
            sl = pl.ds(t * LANES, LANES)
            out_v[sl] = plsc.load_gather(tbl_v, [idx_v[sl]])
            return carry

        lax.fori_loop(0, RW * K // LANES, gcb, 0)
        pltpu.sync_copy(out_v, pay_hbm.at[col, pl.ds(ebase, RW * K)])

    zf = jnp.zeros((LANES,), jnp.float32)

    def zpb(t, carry):
        out_v[pl.ds(t * LANES, LANES)] = zf
        return carry

    lax.fori_loop(0, RW * K // LANES, zpb, 0)
    for col in range(NFC, PAYR):
        pltpu.sync_copy(out_v, pay_hbm.at[col, pl.ds(ebase, RW * K)])


def _nbr_call(xs, ys, ang, mol_t, gen, deg, sang, cang):
    f = pl.kernel(
        _nbr_body,
        out_type=(
            jax.ShapeDtypeStruct((N * K,), jnp.int32),
            jax.ShapeDtypeStruct((N * K,), jnp.float32),
            jax.ShapeDtypeStruct((N * K,), jnp.float32),
            jax.ShapeDtypeStruct((N * K,), jnp.float32),
            jax.ShapeDtypeStruct((PAYR, N * K), jnp.float32),
        ),
        mesh=plsc.VectorSubcoreMesh(core_axis_name="c", subcore_axis_name="s",
                                    num_cores=NC, num_subcores=NS),
        scratch_types=[
            pltpu.VMEM((N,), jnp.float32),
            pltpu.VMEM((N,), jnp.float32),
            pltpu.VMEM((N,), jnp.float32),
            pltpu.VMEM((GR * K,), jnp.int32),
            pltpu.VMEM((GR * K,), jnp.float32),
            pltpu.VMEM((GR * K,), jnp.float32),
            pltpu.VMEM((GR * K,), jnp.float32),
            pltpu.VMEM((RW * K,), jnp.int32),
            pltpu.VMEM((N,), jnp.float32),
            pltpu.VMEM((RW * K,), jnp.float32),
        ],
        compiler_params=pltpu.CompilerParams(use_tc_tiling_on_sc=False,
                                             needs_layout_passes=False),
    )
    return f(xs, ys, ang, mol_t, gen, deg, sang, cang)


DT = 512


def _deg_body(xt_ref, yt_ref, xa_ref, ya_ref, deg_ref):
    xt = xt_ref[...]
    yt = yt_ref[...]
    xa = xa_ref[...]
    ya = ya_ref[...]
    dxm = xt - xa
    dym = yt - ya
    m = (dxm * dxm + dym * dym) <= CUT2
    deg_ref[...] = jnp.sum(m.astype(jnp.float32), axis=1, keepdims=True) - 1.0


def _deg_call(xs, ys):
    return pl.pallas_call(
        _deg_body,
        grid=(N // DT,),
        in_specs=[
            pl.BlockSpec((DT, 1), lambda i: (i, 0)),
            pl.BlockSpec((DT, 1), lambda i: (i, 0)),
            pl.BlockSpec((1, N), lambda i: (0, 0)),
            pl.BlockSpec((1, N), lambda i: (0, 0)),
        ],
        out_specs=pl.BlockSpec((DT, 1), lambda i: (i, 0)),
        out_shape=jax.ShapeDtypeStruct((N, 1), jnp.float32),
    )(xs[:, None], ys[:, None], xs[None, :], ys[None, :])


def _sincos_body(a_ref, s_ref, c_ref):
    a = a_ref[...]
    s_ref[...] = jnp.sin(a)
    c_ref[...] = jnp.cos(a)


def _sincos_call(ang):
    s, c = pl.pallas_call(
        _sincos_body,
        out_shape=(jax.ShapeDtypeStruct((32, 128), jnp.float32),
                   jax.ShapeDtypeStruct((32, 128), jnp.float32)),
    )(ang.reshape(32, 128))
    return s.reshape(N), c.reshape(N)


TN = 32
E = TN * K


def _bc(s, w):
    return lax.broadcast_in_dim(s, (TN, K, w.shape[-1]), (0, 1)) * w[None]


def _dott(a_t, b):
    return lax.dot_general(a_t, b, (((0,), (0,)), ((), ())),
                           preferred_element_type=jnp.float32)


def _tc_body(pay_ref, nf_ref, deg_ref, sang_ref, cang_ref,
             dx_ref, dy_ref, da_ref,
             w1at_ref, w1a_ref, w1s_ref, b1_ref, w2_ref, b2_ref, w3_ref,
             b3_ref, wkvpt_ref, wem2_ref, bkv_ref, wes_ref, wq_ref, bq_ref,
             wskip_ref, bskip_ref, hw1_ref, hb1_ref, hw2_ref, hb2_ref,
             hw3_ref, hb3_ref, out_ref):
    PT = pay_ref[...]
    ni = nf_ref[...]
    dx = dx_ref[...]
    dy = dy_ref[...]
    da = da_ref[...]

    r = jnp.sqrt(jnp.maximum(dx * dx + dy * dy, 1e-12))
    sda = jnp.sin(da)
    cda = jnp.cos(da)

    bf16 = jnp.bfloat16
    PTb = PT.astype(bf16)
    pre = (_dott(PTb, w1at_ref[...]).reshape(TN, K, MH)
           + (b1_ref[...] - ni @ w1a_ref[...])[:, None, :]
           + _bc(dx, w1s_ref[0:1]) + _bc(dy, w1s_ref[1:2])
           + _bc(r, w1s_ref[2:3]) + _bc(sda, w1s_ref[3:4])
           + _bc(cda, w1s_ref[4:5]))
    h = jnp.maximum(pre, 0.0).reshape(E, MH).astype(bf16)
    h = jnp.maximum(lax.dot_general(h, w2_ref[...], (((1,), (0,)), ((), ())),
                                    preferred_element_type=jnp.float32)
                    + b2_ref[...], 0.0).astype(bf16)
    msg = jnp.maximum(lax.dot_general(h, w3_ref[...], (((1,), (0,)), ((), ())),
                                      preferred_element_type=jnp.float32)
                      + b3_ref[...], 0.0)

    sfi = jnp.concatenate(
        [sang_ref[...], cang_ref[...], ni[:, 0:16], ni[:, 16:17],
         deg_ref[...]], axis=-1)
    qi = sfi @ wq_ref[...] + bq_ref[...]
    eself = sfi @ wes_ref[...]

    ib = bkv_ref[...] + jnp.concatenate([eself, eself], axis=-1)
    KV2 = ((_dott(PTb, wkvpt_ref[...])
            + lax.dot_general(msg.astype(bf16), wem2_ref[...],
                              (((1,), (0,)), ((), ())),
                              preferred_element_type=jnp.float32))
           .reshape(TN, K, 4 * UH) + ib[:, None, :])
    ve = KV2[..., 2 * UH:]

    kje = KV2[..., :2 * UH] * qi[:, None, :]
    scale = 1.0 / (float(UH) ** 0.5)
    a_h = [jnp.sum(kje[..., h_ * UH:(h_ + 1) * UH], axis=-1) * scale
           for h_ in range(HEADS)]

    degi = deg_ref[...]
    slot = lax.broadcasted_iota(jnp.int32, (TN, K), 1).astype(jnp.float32)
    valid = slot < jnp.minimum(degi, float(K))

    outs = []
    for h_ in range(HEADS):
        ah = jnp.where(valid, a_h[h_], -1e30)
        am = jnp.max(ah, axis=1, keepdims=True)
        am = jnp.where(degi > 0.0, am, 0.0)
        ea = jnp.where(valid, jnp.exp(ah - am), 0.0)
        den = jnp.sum(ea, axis=1, keepdims=True)
        inv = 1.0 / (den + 1e-16)
        eab = lax.broadcast_in_dim(ea, (TN, K, UH), (0, 1))
        oh = jnp.sum(ve[..., h_ * UH:(h_ + 1) * UH] * eab, axis=1)
        outs.append(oh * inv)

    out = (outs[0] + outs[1]) * 0.5 + sfi @ wskip_ref[...] + bskip_ref[...]
    hh = jnp.maximum(out @ hw1_ref[...] + hb1_ref[...], 0.0)
    hh = jnp.maximum(hh @ hw2_ref[...] + hb2_ref[...], 0.0)
    out_ref[...] = hh @ hw3_ref[...] + hb3_ref[...]


def _tc_call(payload_t, nodefeat0, deg, sang, cang, dxp, dyp, dap, *weights):
    wspecs = [pl.BlockSpec(w.shape, lambda i, nd=w.ndim: (0,) * nd)
              for w in weights]
    return pl.pallas_call(
        _tc_body,
        grid=(N // TN,),
        in_specs=[
            pl.BlockSpec((PAYR, E), lambda i: (0, i)),
            pl.BlockSpec((TN, 24), lambda i: (i, 0)),
            pl.BlockSpec((TN, 1), lambda i: (i, 0)),
            pl.BlockSpec((TN, 1), lambda i: (i, 0)),
            pl.BlockSpec((TN, 1), lambda i: (i, 0)),
            pl.BlockSpec((TN, K), lambda i: (i, 0)),
            pl.BlockSpec((TN, K), lambda i: (i, 0)),
            pl.BlockSpec((TN, K), lambda i: (i, 0)),
            *wspecs,
        ],
        out_specs=pl.BlockSpec((TN, PAY), lambda i: (i, 0)),
        out_shape=jax.ShapeDtypeStruct((N, PAY), jnp.float32),
    )(payload_t, nodefeat0, deg, sang, cang, dxp, dyp, dap, *weights)


def kernel(x, angle, molecules, generation, msg_W1, msg_b1, msg_W2, msg_b2,
           msg_W3, msg_b3, Wq, bq, Wk, bk, Wv, bv, We, Wskip, bskip,
           head_W1, head_b1, head_W2, head_b2, head_W3, head_b3):
    f32 = jnp.float32
    xs = jnp.asarray(x[:, 0], f32)
    ys = jnp.asarray(x[:, 1], f32)
    ang = jnp.asarray(angle[:, 0], f32)

    deg2 = _deg_call(xs, ys)
    deg = deg2[:, 0]
    sang, cang = _sincos_call(ang)
    mol_t = molecules.T.reshape(-1)
    gen = generation[:, 0]

    nbr_flat, dxf, dyf, daf, payload_t = _nbr_call(
        xs, ys, ang, mol_t, gen, deg, sang, cang)

    nodefeat0 = jnp.concatenate(
        [molecules, generation, jnp.zeros((N, 7), f32)], axis=1)

    w1at = jnp.zeros((PAYR, MH), f32).at[0:16].set(msg_W1[5:21])
    w1a = jnp.zeros((24, MH), f32).at[0:16].set(msg_W1[5:21])
    w1s = jnp.zeros((8, MH), f32).at[0:5].set(msg_W1[0:5])
    wkv = jnp.concatenate([Wk, Wv], axis=1)
    wkvpt = jnp.zeros((PAYR, 2 * HU), f32).at[0:16].set(wkv[2:18])
    wkvpt = (wkvpt.at[16].set(wkv[18]).at[17].set(wkv[19])
             .at[18].set(wkv[0]).at[19].set(wkv[1]))
    bkv = jnp.concatenate([bk, bv])[None, :]
    wem = We[:MH]
    wem2 = jnp.concatenate([wem, wem], axis=1)
    wes = We[MH:]
    hw3p = jnp.concatenate([head_W3, jnp.zeros((UH, PAY - OUT_DIM), f32)],
                           axis=1)
    hb3p = jnp.concatenate([head_b3, jnp.zeros((PAY - OUT_DIM,), f32)])[None, :]

    bf16 = jnp.bfloat16
    upd = _tc_call(payload_t, nodefeat0, deg2, sang[:, None],
                   cang[:, None],
                   dxf.reshape(N, K), dyf.reshape(N, K), daf.reshape(N, K),
                   w1at.astype(bf16), w1a, w1s, msg_b1[None, :],
                   msg_W2.astype(bf16), msg_b2[None, :],
                   msg_W3.astype(bf16), msg_b3[None, :],
                   wkvpt.astype(bf16), wem2.astype(bf16), bkv, wes,
                   Wq, bq[None, :], Wskip, bskip[None, :],
                   head_W1, head_b1[None, :], head_W2, head_b2[None, :],
                   hw3p, hb3p)

    return (upd[:, 0:2], upd[:, 2:3], upd[:, 3:3 + MOL],
            upd[:, 3 + MOL:4 + MOL])
